# Initial kernel scaffold; baseline (speedup 1.0000x reference)
import jax
import jax.numpy as jnp
from jax.experimental import pallas as pl


def kernel(x_bus, x_gen, x_ext, params, edge_index_bus__line__bus, edge_attr_bus__line__bus, edge_index_gen__conn__bus, edge_attr_gen__conn__bus, edge_index_ext__conn__bus, edge_attr_ext__conn__bus, edge_index_bus__conn__gen, edge_attr_bus__conn__gen, edge_index_bus__conn__ext, edge_attr_bus__conn__ext):
    raise NotImplementedError("write your pallas kernel here")


# speedup vs baseline: 1.8125x; 1.8125x over previous
import functools

import jax
import jax.numpy as jnp
from jax import lax
from jax.experimental import pallas as pl
from jax.experimental.pallas import tpu as pltpu
from jax.experimental.pallas import tpu_sc as plsc

_H = 64
_HH = 32
_CHUNK = 128
_NTILES = 16
_EBLK = _CHUNK * _NTILES
_ZR = 128


@functools.lru_cache(maxsize=None)
def _sc_msg(n_src, n_dst, e_pad, ndp):
    nchunks = e_pad // _EBLK
    n_acc = -(-(n_dst + 8) // (_NTILES * _ZR)) * (_NTILES * _ZR)
    zdmas = n_acc // (_NTILES * _ZR)
    wbp = ndp // _NTILES
    mesh = plsc.VectorSubcoreMesh(core_axis_name="c", subcore_axis_name="s")

    @functools.partial(
        pl.kernel,
        out_type=jax.ShapeDtypeStruct((2, ndp, _HH), jnp.float32),
        mesh=mesh,
        compiler_params=pltpu.CompilerParams(use_tc_tiling_on_sc=False),
        scratch_types=[
            pltpu.VMEM_SHARED((n_acc, _HH), jnp.float32),
            pltpu.VMEM((_CHUNK,), jnp.int32),
            pltpu.VMEM((_CHUNK,), jnp.int32),
            pltpu.VMEM((_CHUNK, _HH), jnp.float32),
            pltpu.VMEM((_CHUNK, _HH), jnp.float32),
            pltpu.VMEM((_ZR, _HH), jnp.float32),
            pltpu.SemaphoreType.DMA,
            pltpu.SemaphoreType.DMA,
        ],
    )
    def k(x_hbm, e_hbm, src_hbm, dst_hbm, out_hbm,
          acc, sidx, didx, xg, eb, zb, sem1, sem2):
        c = lax.axis_index("c")
        s = lax.axis_index("s")

        zv = jnp.zeros((16,), jnp.float32)

        def zrow(i, carry):
            zb[i, pl.ds(0, 16)] = zv
            zb[i, pl.ds(16, 16)] = zv
            return carry

        lax.fori_loop(0, _ZR, zrow, 0, unroll=8)

        def zdma(i, carry):
            pltpu.sync_copy(zb, acc.at[pl.ds((s * zdmas + i) * _ZR, _ZR)])
            return carry

        lax.fori_loop(0, zdmas, zdma, 0)
        plsc.subcore_barrier()

        def chunk(j, carry):
            off = (s * nchunks + j) * _CHUNK
            pltpu.sync_copy(src_hbm.at[pl.ds(off, _CHUNK)], sidx)
            pltpu.sync_copy(dst_hbm.at[pl.ds(off, _CHUNK)], didx)
            cp1 = pltpu.async_copy(x_hbm.at[c].at[sidx], xg, sem1)
            cp2 = pltpu.async_copy(e_hbm.at[c].at[pl.ds(off, _CHUNK)], eb, sem2)
            cp1.wait()
            cp2.wait()

            def crow(r, carry2):
                xg[r, pl.ds(0, 16)] = jnp.maximum(
                    xg[r, pl.ds(0, 16)] + eb[r, pl.ds(0, 16)], 0.0)
                xg[r, pl.ds(16, 16)] = jnp.maximum(
                    xg[r, pl.ds(16, 16)] + eb[r, pl.ds(16, 16)], 0.0)
                return carry2

            lax.fori_loop(0, _CHUNK, crow, 0, unroll=4)
            pltpu.sync_copy(xg, acc.at[didx], add=True)
            return carry

        lax.fori_loop(0, nchunks, chunk, 0)
        plsc.subcore_barrier()
        pltpu.sync_copy(acc.at[pl.ds(s * wbp, wbp)],
                        out_hbm.at[c].at[pl.ds(s * wbp, wbp)])

    return k


def _proj_body(x_ref, w_ref, b_ref, o_ref):
    y = jnp.dot(x_ref[...], w_ref[...],
                preferred_element_type=jnp.float32) + b_ref[...]
    o_ref[0] = y[:, :_HH]
    o_ref[1] = y[:, _HH:]


def _proj(x, w, b):
    n, d = x.shape
    blk = min(2048, -(-n // 8) * 8)
    grid = -(-n // blk)
    return pl.pallas_call(
        _proj_body,
        grid=(grid,),
        in_specs=[pl.BlockSpec((blk, d), lambda i: (i, 0)),
                  pl.BlockSpec((d, _H), lambda i: (0, 0)),
                  pl.BlockSpec((_H,), lambda i: (0,))],
        out_specs=pl.BlockSpec((2, blk, _HH), lambda i: (0, i, 0)),
        out_shape=jax.ShapeDtypeStruct((2, n, _HH), jnp.float32),
    )(x, w, b)


def _e3_body(a_ref, w_ref, b_ref, o0, o1, o2):
    a = a_ref[...]
    for l, o in enumerate((o0, o1, o2)):
        y = jnp.dot(a, w_ref[l], preferred_element_type=jnp.float32) + b_ref[l]
        o[0] = y[:, :_HH]
        o[1] = y[:, _HH:]


def _e3(ea, wl, bl):
    ep, d = ea.shape
    blk = 2048
    grid = ep // blk
    ot = jax.ShapeDtypeStruct((2, ep, _HH), jnp.float32)
    return pl.pallas_call(
        _e3_body,
        grid=(grid,),
        in_specs=[pl.BlockSpec((blk, d), lambda i: (i, 0)),
                  pl.BlockSpec((3, d, _H), lambda i: (0, 0, 0)),
                  pl.BlockSpec((3, _H), lambda i: (0, 0))],
        out_specs=[pl.BlockSpec((2, blk, _HH), lambda i: (0, i, 0))] * 3,
        out_shape=[ot, ot, ot],
    )(ea, wl, bl)


def _post_body(nt, x_ref, *rest):
    a_refs = rest[:nt]
    w1_ref, b1_ref, w2_ref, b2_ref, o_ref = rest[nt:]
    xlo = x_ref[0]
    xhi = x_ref[1]
    acc = None
    for t in range(nt):
        hlo = xlo + a_refs[t][0]
        hhi = xhi + a_refs[t][1]
        h = (jnp.dot(hlo, w1_ref[t, :_HH, :], preferred_element_type=jnp.float32)
             + jnp.dot(hhi, w1_ref[t, _HH:, :], preferred_element_type=jnp.float32)
             + b1_ref[t])
        h = jnp.maximum(h, 0.0)
        y = jnp.dot(h, w2_ref[t], preferred_element_type=jnp.float32) + b2_ref[t]
        y = jnp.maximum(y, 0.0)
        acc = y if acc is None else acc + y
    y = acc / nt
    y = jnp.where(y > 0, y, 0.2 * y)
    o_ref[0] = y[:, :_HH]
    o_ref[1] = y[:, _HH:]


def _post(x, aggs, w1s, b1s, w2s, b2s):
    nt = len(aggs)
    n = x.shape[1]
    blk = min(1024, -(-n // 8) * 8)
    grid = -(-n // blk)
    return pl.pallas_call(
        functools.partial(_post_body, nt),
        grid=(grid,),
        in_specs=([pl.BlockSpec((2, blk, _HH), lambda i: (0, i, 0))] * (1 + nt)
                  + [pl.BlockSpec((nt, _H, _H), lambda i: (0, 0, 0)),
                     pl.BlockSpec((nt, _H), lambda i: (0, 0)),
                     pl.BlockSpec((nt, _H, _H), lambda i: (0, 0, 0)),
                     pl.BlockSpec((nt, _H), lambda i: (0, 0))]),
        out_specs=pl.BlockSpec((2, blk, _HH), lambda i: (0, i, 0)),
        out_shape=jax.ShapeDtypeStruct((2, n, _HH), jnp.float32),
    )(x, *aggs, w1s, b1s, w2s, b2s)


def _head_body(x1, x2, x3, w0, b0, w1, b1, wf, bf, o_ref):
    jlo = (x1[0] + x2[0] + x3[0]) / 3.0
    jhi = (x1[1] + x2[1] + x3[1]) / 3.0
    rlo = jnp.maximum(jlo, 0.0)
    rhi = jnp.maximum(jhi, 0.0)
    h = (jnp.dot(rlo, w0[:_HH, :], preferred_element_type=jnp.float32)
         + jnp.dot(rhi, w0[_HH:, :], preferred_element_type=jnp.float32)
         + b0[...])
    h = jnp.maximum(h, 0.0)
    h = jnp.dot(h, w1[...], preferred_element_type=jnp.float32) + b1[...]
    o_ref[...] = jnp.dot(h, wf[...], preferred_element_type=jnp.float32) + bf[...]


def _head(x1, x2, x3, w0, b0, w1, b1, wf, bf):
    n = x1.shape[1]
    odim = wf.shape[1]
    blk = min(1024, -(-n // 8) * 8)
    grid = -(-n // blk)
    return pl.pallas_call(
        _head_body,
        grid=(grid,),
        in_specs=([pl.BlockSpec((2, blk, _HH), lambda i: (0, i, 0))] * 3
                  + [pl.BlockSpec((_H, _H), lambda i: (0, 0)),
                     pl.BlockSpec((_H,), lambda i: (0,)),
                     pl.BlockSpec((_H, _H), lambda i: (0, 0)),
                     pl.BlockSpec((_H,), lambda i: (0,)),
                     pl.BlockSpec((_H, odim), lambda i: (0, 0)),
                     pl.BlockSpec((odim,), lambda i: (0,))]),
        out_specs=pl.BlockSpec((blk, odim), lambda i: (i, 0)),
        out_shape=jax.ShapeDtypeStruct((n, odim), jnp.float32),
    )(x1, x2, x3, w0, b0, w1, b1, wf, bf)


def kernel(x_bus, x_gen, x_ext, params,
           edge_index_bus__line__bus, edge_attr_bus__line__bus,
           edge_index_gen__conn__bus, edge_attr_gen__conn__bus,
           edge_index_ext__conn__bus, edge_attr_ext__conn__bus,
           edge_index_bus__conn__gen, edge_attr_bus__conn__gen,
           edge_index_bus__conn__ext, edge_attr_bus__conn__ext):
    ets = [
        ("bus", "bus", "bus__line__bus",
         edge_index_bus__line__bus, edge_attr_bus__line__bus),
        ("gen", "bus", "gen__conn__bus",
         edge_index_gen__conn__bus, edge_attr_gen__conn__bus),
        ("ext", "bus", "ext__conn__bus",
         edge_index_ext__conn__bus, edge_attr_ext__conn__bus),
        ("bus", "gen", "bus__conn__gen",
         edge_index_bus__conn__gen, edge_attr_bus__conn__gen),
        ("bus", "ext", "bus__conn__ext",
         edge_index_bus__conn__ext, edge_attr_bus__conn__ext),
    ]
    nn = {"bus": x_bus.shape[0], "gen": x_gen.shape[0], "ext": x_ext.shape[0]}
    ndp = {k: -(-v // (8 * _NTILES)) * (8 * _NTILES) for k, v in nn.items()}

    prep = []
    for src_t, dst_t, name, ei, ea in ets:
        e = ea.shape[0]
        ep = -(-e // _EBLK) * _EBLK
        src = jnp.concatenate(
            [ei[0].astype(jnp.int32), jnp.zeros((ep - e,), jnp.int32)])
        dst = jnp.concatenate(
            [ei[1].astype(jnp.int32),
             jnp.full((ep - e,), nn[dst_t], jnp.int32)])
        eap = jnp.concatenate(
            [ea, jnp.zeros((ep - e, ea.shape[1]), ea.dtype)])
        prep.append((src_t, dst_t, name, ep, src, dst, eap))

    x = {}
    for nt, xv in (("bus", x_bus), ("gen", x_gen), ("ext", x_ext)):
        p = params["in"][nt]
        x[nt] = _proj(xv, p["W"], p["b"])

    e_all = {}
    for src_t, dst_t, name, ep, src, dst, eap in prep:
        wl = jnp.stack([params["convs"][l][name]["edge"]["W"] for l in range(3)])
        bl = jnp.stack([params["convs"][l][name]["edge"]["b"] for l in range(3)])
        e_all[name] = _e3(eap, wl, bl)

    jk = {"bus": [], "gen": []}
    for l in range(3):
        aggs = {"bus": [], "gen": [], "ext": []}
        for src_t, dst_t, name, ep, src, dst, eap in prep:
            agg = _sc_msg(nn[src_t], nn[dst_t], ep, ndp[dst_t])(
                x[src_t], e_all[name][l], src, dst)
            aggs[dst_t].append(agg)
        newx = {}
        for nt in ("bus", "gen", "ext"):
            names = [name for (s2, d2, name, *_r) in prep if d2 == nt]
            w1s, b1s, w2s, b2s = [], [], [], []
            for name in names:
                p = params["convs"][l][name]
                s_bn = p["bn_g"] / jnp.sqrt(p["bn_v"] + 1e-5)
                t_bn = p["bn_b"] - p["bn_m"] * s_bn
                w1s.append(p["lin1"]["W"] * s_bn[None, :])
                b1s.append(p["lin1"]["b"] * s_bn + t_bn)
                w2s.append(p["lin2"]["W"])
                b2s.append(p["lin2"]["b"])
            newx[nt] = _post(x[nt], aggs[nt], jnp.stack(w1s), jnp.stack(b1s),
                             jnp.stack(w2s), jnp.stack(b2s))
            if nt in jk:
                jk[nt].append(newx[nt])
        x = newx

    outs = []
    for nt in ("bus", "gen"):
        ps = params["lins"]
        outs.append(_head(jk[nt][0], jk[nt][1], jk[nt][2],
                          ps[0][nt]["W"], ps[0][nt]["b"],
                          ps[1][nt]["W"], ps[1][nt]["b"],
                          ps[2][nt]["W"], ps[2][nt]["b"]))
    return tuple(outs)


# speedup vs baseline: 1.8459x; 1.0184x over previous
import functools

import jax
import jax.numpy as jnp
from jax import lax
from jax.experimental import pallas as pl
from jax.experimental.pallas import tpu as pltpu
from jax.experimental.pallas import tpu_sc as plsc

_H = 64
_HH = 32
_CHUNK = 128
_NTILES = 16
_GRP = 4
_EBLK = _CHUNK * _NTILES * _GRP * 2
_ACC_ROWS = 53248


def _phase(refs, c, s, x_hbm, ea_hbm, s2_hbm, d2_hbm, out_hbm,
           nc, ndp, tslot):
    (acc, sidx, didx, xg, sb, eab, wv, bv,
     semi, semg, sems) = refs
    wbp = ndp // _NTILES

    w_vecs = [[wv[pl.ds(j * _H + c * _HH + v * 16, 16)] for v in (0, 1)]
              for j in range(4)]
    b_vecs = [bv[pl.ds(c * _HH + v * 16, 16)] for v in (0, 1)]

    zv = jnp.zeros((16,), jnp.float32)

    def zrow(i, carry):
        sb[0][i, pl.ds(0, 16)] = zv
        sb[0][i, pl.ds(16, 16)] = zv
        return carry

    lax.fori_loop(0, _CHUNK, zrow, 0, unroll=8)

    zfull = wbp // _CHUNK
    ztail = wbp % _CHUNK

    def zdma(i, carry):
        pltpu.async_copy(sb[0], acc.at[pl.ds(s * wbp + i * _CHUNK, _CHUNK)],
                         semi[0])
        return carry

    lax.fori_loop(0, zfull, zdma, 0)
    if ztail:
        pltpu.async_copy(sb[0].at[pl.ds(0, ztail)],
                         acc.at[pl.ds(s * wbp + zfull * _CHUNK, ztail)],
                         semi[0])

    def zdrain(i, carry):
        pltpu.make_async_copy(
            sb[0], acc.at[pl.ds(s * wbp, _CHUNK)], semi[0]).wait()
        return carry

    lax.fori_loop(0, zfull, zdrain, 0)
    if ztail:
        pltpu.make_async_copy(
            sb[0].at[pl.ds(0, ztail)], acc.at[pl.ds(s * wbp, ztail)],
            semi[0]).wait()
    plsc.subcore_barrier()

    base = s * nc

    def idx_issue(g_next, gs):
        pltpu.async_copy(s2_hbm.at[pl.ds(base + g_next * _GRP, _GRP)],
                         sidx.at[gs], semi[gs])
        pltpu.async_copy(d2_hbm.at[pl.ds(base + g_next * _GRP, _GRP)],
                         didx.at[gs], semi[gs])

    def idx_wait(gs):
        pltpu.make_async_copy(s2_hbm.at[pl.ds(base, _GRP)],
                              sidx.at[gs], semi[gs]).wait()
        pltpu.make_async_copy(d2_hbm.at[pl.ds(base, _GRP)],
                              didx.at[gs], semi[gs]).wait()

    def gather_issue(j, slot, gs, kk):
        pltpu.async_copy(x_hbm.at[c].at[sidx.at[gs].at[kk]], xg[slot],
                         semg[slot])
        pltpu.async_copy(ea_hbm.at[pl.ds((base + j) * (_CHUNK * 4),
                                         _CHUNK * 4)],
                         eab[slot], semg[slot])

    def gather_wait(j, slot, gs, kk):
        pltpu.make_async_copy(x_hbm.at[c].at[sidx.at[gs].at[kk]], xg[slot],
                              semg[slot]).wait()
        pltpu.make_async_copy(ea_hbm.at[pl.ds(base * (_CHUNK * 4),
                                              _CHUNK * 4)],
                              eab[slot], semg[slot]).wait()

    def scatter_issue(slot, gs, kk):
        pltpu.async_copy(sb[slot], acc.at[didx.at[gs].at[kk]], sems[slot],
                         add=True)

    def scatter_drain(slot):
        pltpu.make_async_copy(sb[slot], acc.at[didx.at[0].at[0]],
                              sems[slot]).wait()

    def compute(slot):
        xgs = sb[slot]
        xsrc = xg[slot]
        eas = eab[slot]

        def cgrp(i, carry):
            va = eas[pl.ds(16 * i, 16)]
            for e4 in range(4):
                r = 4 * i + e4
                for v in (0, 1):
                    ev = (b_vecs[v]
                          + va[4 * e4] * w_vecs[0][v]
                          + va[4 * e4 + 1] * w_vecs[1][v]
                          + va[4 * e4 + 2] * w_vecs[2][v]
                          + va[4 * e4 + 3] * w_vecs[3][v])
                    xgs[r, pl.ds(16 * v, 16)] = jnp.maximum(
                        xsrc[r, pl.ds(16 * v, 16)] + ev, 0.0)
            return carry

        lax.fori_loop(0, _CHUNK // 4, cgrp, 0, unroll=1)

    ngrp = nc // _GRP
    npair = ngrp // 2

    pltpu.sync_copy(s2_hbm.at[pl.ds(base, _GRP)], sidx.at[0])
    pltpu.sync_copy(d2_hbm.at[pl.ds(base, _GRP)], didx.at[0])
    gather_issue(0, 0, 0, 0)

    def pair(p, carry):
        for gh in (0, 1):
            g = p * 2 + gh
            for kk in range(_GRP):
                j = g * _GRP + kk
                slot = kk % 2
                nslot = (kk + 1) % 2
                if kk < _GRP - 1:
                    gather_issue(j + 1, nslot, gh, kk + 1)
                else:
                    @pl.when(g + 1 < ngrp)
                    def _():
                        idx_wait(gh ^ 1)
                        gather_issue(j + 1, nslot, gh ^ 1, 0)
                gather_wait(j, slot, gh, kk)

                @pl.when(j >= 2)
                def _():
                    scatter_drain(slot)
                if kk == 2:
                    @pl.when(g + 1 < ngrp)
                    def _():
                        idx_issue(g + 1, gh ^ 1)
                compute(slot)
                scatter_issue(slot, gh, kk)
        return carry

    lax.fori_loop(0, npair, pair, 0)
    scatter_drain(0)
    scatter_drain(1)
    plsc.subcore_barrier()
    pltpu.sync_copy(acc.at[pl.ds(s * wbp, wbp)],
                    out_hbm.at[c].at[pl.ds(s * wbp, wbp)])
    plsc.subcore_barrier()


@functools.lru_cache(maxsize=None)
def _sc_layer(type_meta):
    mesh = plsc.VectorSubcoreMesh(core_axis_name="c", subcore_axis_name="s")
    out_types = [jax.ShapeDtypeStruct((2, ndp, _HH), jnp.float32)
                 for (_xs, _ep, ndp) in type_meta]

    @functools.partial(
        pl.kernel,
        out_type=out_types,
        mesh=mesh,
        compiler_params=pltpu.CompilerParams(use_tc_tiling_on_sc=False),
        scratch_types=[
            pltpu.VMEM_SHARED((_ACC_ROWS, _HH), jnp.float32),
            pltpu.VMEM((2, _GRP, _CHUNK), jnp.int32),
            pltpu.VMEM((2, _GRP, _CHUNK), jnp.int32),
            pltpu.VMEM((_CHUNK, _HH), jnp.float32),
            pltpu.VMEM((_CHUNK, _HH), jnp.float32),
            pltpu.VMEM((_CHUNK, _HH), jnp.float32),
            pltpu.VMEM((_CHUNK, _HH), jnp.float32),
            pltpu.VMEM((_CHUNK * 4,), jnp.float32),
            pltpu.VMEM((_CHUNK * 4,), jnp.float32),
            pltpu.VMEM((4 * _H,), jnp.float32),
            pltpu.VMEM((_H,), jnp.float32),
            pltpu.SemaphoreType.DMA,
            pltpu.SemaphoreType.DMA,
            pltpu.SemaphoreType.DMA,
            pltpu.SemaphoreType.DMA,
            pltpu.SemaphoreType.DMA,
            pltpu.SemaphoreType.DMA,
        ],
    )
    def k(*args):
        nt = len(type_meta)
        xs = args[0:3]
        per = args[3:3 + 3 * nt]
        wflat = args[3 + 3 * nt]
        bflat = args[4 + 3 * nt]
        outs = args[5 + 3 * nt:5 + 4 * nt]
        (acc, sidx, didx, xg0, xg1, sb0, sb1, ea0, ea1, wv, bv,
         si0, si1, sg0, sg1, ss0, ss1) = args[5 + 4 * nt:]
        c = lax.axis_index("c")
        s = lax.axis_index("s")
        for t, (x_slot, ep, ndp) in enumerate(type_meta):
            ea_hbm = per[3 * t]
            s2_hbm = per[3 * t + 1]
            d2_hbm = per[3 * t + 2]
            pltpu.sync_copy(wflat.at[pl.ds(t * 4 * _H, 4 * _H)], wv)
            pltpu.sync_copy(bflat.at[pl.ds(t * _H, _H)], bv)
            refs = (acc, sidx, didx, (xg0, xg1), (sb0, sb1), (ea0, ea1),
                    wv, bv, (si0, si1), (sg0, sg1), (ss0, ss1))
            _phase(refs, c, s, xs[x_slot], ea_hbm, s2_hbm, d2_hbm,
                   outs[t], ep // (_CHUNK * _NTILES), ndp, t)

    return k


def _proj_body(x_ref, w_ref, b_ref, o_ref):
    y = jnp.dot(x_ref[...], w_ref[...],
                preferred_element_type=jnp.float32) + b_ref[...]
    o_ref[0] = y[:, :_HH]
    o_ref[1] = y[:, _HH:]


def _proj(x, w, b):
    n, d = x.shape
    blk = min(2048, -(-n // 8) * 8)
    grid = -(-n // blk)
    return pl.pallas_call(
        _proj_body,
        grid=(grid,),
        in_specs=[pl.BlockSpec((blk, d), lambda i: (i, 0)),
                  pl.BlockSpec((d, _H), lambda i: (0, 0)),
                  pl.BlockSpec((_H,), lambda i: (0,))],
        out_specs=pl.BlockSpec((2, blk, _HH), lambda i: (0, i, 0)),
        out_shape=jax.ShapeDtypeStruct((2, n, _HH), jnp.float32),
    )(x, w, b)


def _post_body(nt, x_ref, *rest):
    a_refs = rest[:nt]
    w1_ref, b1_ref, w2_ref, b2_ref, o_ref = rest[nt:]
    xlo = x_ref[0]
    xhi = x_ref[1]
    acc = None
    for t in range(nt):
        hlo = xlo + a_refs[t][0]
        hhi = xhi + a_refs[t][1]
        h = (jnp.dot(hlo, w1_ref[t, :_HH, :], preferred_element_type=jnp.float32)
             + jnp.dot(hhi, w1_ref[t, _HH:, :], preferred_element_type=jnp.float32)
             + b1_ref[t])
        h = jnp.maximum(h, 0.0)
        y = jnp.dot(h, w2_ref[t], preferred_element_type=jnp.float32) + b2_ref[t]
        y = jnp.maximum(y, 0.0)
        acc = y if acc is None else acc + y
    y = acc / nt
    y = jnp.where(y > 0, y, 0.2 * y)
    o_ref[0] = y[:, :_HH]
    o_ref[1] = y[:, _HH:]


def _post(x, aggs, w1s, b1s, w2s, b2s):
    nt = len(aggs)
    n = x.shape[1]
    blk = min(1024, -(-n // 8) * 8)
    grid = -(-n // blk)
    return pl.pallas_call(
        functools.partial(_post_body, nt),
        grid=(grid,),
        in_specs=([pl.BlockSpec((2, blk, _HH), lambda i: (0, i, 0))] * (1 + nt)
                  + [pl.BlockSpec((nt, _H, _H), lambda i: (0, 0, 0)),
                     pl.BlockSpec((nt, _H), lambda i: (0, 0)),
                     pl.BlockSpec((nt, _H, _H), lambda i: (0, 0, 0)),
                     pl.BlockSpec((nt, _H), lambda i: (0, 0))]),
        out_specs=pl.BlockSpec((2, blk, _HH), lambda i: (0, i, 0)),
        out_shape=jax.ShapeDtypeStruct((2, n, _HH), jnp.float32),
    )(x, *aggs, w1s, b1s, w2s, b2s)


def _head_body(x1, x2, x3, w0, b0, w1, b1, wf, bf, o_ref):
    jlo = (x1[0] + x2[0] + x3[0]) / 3.0
    jhi = (x1[1] + x2[1] + x3[1]) / 3.0
    rlo = jnp.maximum(jlo, 0.0)
    rhi = jnp.maximum(jhi, 0.0)
    h = (jnp.dot(rlo, w0[:_HH, :], preferred_element_type=jnp.float32)
         + jnp.dot(rhi, w0[_HH:, :], preferred_element_type=jnp.float32)
         + b0[...])
    h = jnp.maximum(h, 0.0)
    h = jnp.dot(h, w1[...], preferred_element_type=jnp.float32) + b1[...]
    o_ref[...] = jnp.dot(h, wf[...], preferred_element_type=jnp.float32) + bf[...]


def _head(x1, x2, x3, w0, b0, w1, b1, wf, bf):
    n = x1.shape[1]
    odim = wf.shape[1]
    blk = min(1024, -(-n // 8) * 8)
    grid = -(-n // blk)
    return pl.pallas_call(
        _head_body,
        grid=(grid,),
        in_specs=([pl.BlockSpec((2, blk, _HH), lambda i: (0, i, 0))] * 3
                  + [pl.BlockSpec((_H, _H), lambda i: (0, 0)),
                     pl.BlockSpec((_H,), lambda i: (0,)),
                     pl.BlockSpec((_H, _H), lambda i: (0, 0)),
                     pl.BlockSpec((_H,), lambda i: (0,)),
                     pl.BlockSpec((_H, odim), lambda i: (0, 0)),
                     pl.BlockSpec((odim,), lambda i: (0,))]),
        out_specs=pl.BlockSpec((blk, odim), lambda i: (i, 0)),
        out_shape=jax.ShapeDtypeStruct((n, odim), jnp.float32),
    )(x1, x2, x3, w0, b0, w1, b1, wf, bf)


def kernel(x_bus, x_gen, x_ext, params,
           edge_index_bus__line__bus, edge_attr_bus__line__bus,
           edge_index_gen__conn__bus, edge_attr_gen__conn__bus,
           edge_index_ext__conn__bus, edge_attr_ext__conn__bus,
           edge_index_bus__conn__gen, edge_attr_bus__conn__gen,
           edge_index_bus__conn__ext, edge_attr_bus__conn__ext):
    ets = [
        ("bus", "bus", "bus__line__bus",
         edge_index_bus__line__bus, edge_attr_bus__line__bus),
        ("gen", "bus", "gen__conn__bus",
         edge_index_gen__conn__bus, edge_attr_gen__conn__bus),
        ("ext", "bus", "ext__conn__bus",
         edge_index_ext__conn__bus, edge_attr_ext__conn__bus),
        ("bus", "gen", "bus__conn__gen",
         edge_index_bus__conn__gen, edge_attr_bus__conn__gen),
        ("bus", "ext", "bus__conn__ext",
         edge_index_bus__conn__ext, edge_attr_bus__conn__ext),
    ]
    xslot = {"bus": 0, "gen": 1, "ext": 2}
    nn = {"bus": x_bus.shape[0], "gen": x_gen.shape[0], "ext": x_ext.shape[0]}
    ndp = {k: -(-v // 128) * 128 for k, v in nn.items()}

    prep = []
    for src_t, dst_t, name, ei, ea in ets:
        e = ea.shape[0]
        ep = -(-e // _EBLK) * _EBLK
        src = jnp.concatenate(
            [ei[0].astype(jnp.int32), jnp.zeros((ep - e,), jnp.int32)])
        dst = jnp.concatenate(
            [ei[1].astype(jnp.int32),
             jnp.full((ep - e,), nn[dst_t], jnp.int32)])
        eaf = jnp.concatenate(
            [ea, jnp.zeros((ep - e, ea.shape[1]), ea.dtype)]).reshape(-1)
        prep.append((src_t, dst_t, name, ep,
                     src.reshape(-1, _CHUNK), dst.reshape(-1, _CHUNK), eaf))

    x = {}
    for nt, xv in (("bus", x_bus), ("gen", x_gen), ("ext", x_ext)):
        p = params["in"][nt]
        x[nt] = _proj(xv, p["W"], p["b"])

    type_meta = tuple((xslot[src_t], ep, ndp[dst_t])
                      for (src_t, dst_t, name, ep, _s, _d, _e) in prep)
    sc_call = _sc_layer(type_meta)

    jk = {"bus": [], "gen": []}
    for l in range(3):
        wflat = jnp.concatenate(
            [params["convs"][l][name]["edge"]["W"].reshape(-1)
             for (_s2, _d2, name, *_r) in prep])
        bflat = jnp.concatenate(
            [params["convs"][l][name]["edge"]["b"]
             for (_s2, _d2, name, *_r) in prep])
        per_args = []
        for (_src_t, _dst_t, _name, _ep, s2, d2, eaf) in prep:
            per_args += [eaf, s2, d2]
        outs_l = sc_call(x["bus"], x["gen"], x["ext"], *per_args,
                         wflat, bflat)
        aggs = {"bus": [], "gen": [], "ext": []}
        for (src_t, dst_t, name, *_r), agg in zip(prep, outs_l):
            aggs[dst_t].append(agg)
        newx = {}
        for nt in ("bus", "gen", "ext"):
            names = [name for (s2_, d2_, name, *_r) in prep if d2_ == nt]
            w1s, b1s, w2s, b2s = [], [], [], []
            for name in names:
                p = params["convs"][l][name]
                s_bn = p["bn_g"] / jnp.sqrt(p["bn_v"] + 1e-5)
                t_bn = p["bn_b"] - p["bn_m"] * s_bn
                w1s.append(p["lin1"]["W"] * s_bn[None, :])
                b1s.append(p["lin1"]["b"] * s_bn + t_bn)
                w2s.append(p["lin2"]["W"])
                b2s.append(p["lin2"]["b"])
            newx[nt] = _post(x[nt], aggs[nt], jnp.stack(w1s), jnp.stack(b1s),
                             jnp.stack(w2s), jnp.stack(b2s))
            if nt in jk:
                jk[nt].append(newx[nt])
        x = newx

    outs = []
    for nt in ("bus", "gen"):
        ps = params["lins"]
        outs.append(_head(jk[nt][0], jk[nt][1], jk[nt][2],
                          ps[0][nt]["W"], ps[0][nt]["b"],
                          ps[1][nt]["W"], ps[1][nt]["b"],
                          ps[2][nt]["W"], ps[2][nt]["b"]))
    return tuple(outs)
